# Initial kernel scaffold; baseline (speedup 1.0000x reference)
import jax
import jax.numpy as jnp
from jax.experimental import pallas as pl


def kernel(x, edge_index, pos, W, b, head_w, head_b):
    raise NotImplementedError("write your pallas kernel here")


# speedup vs baseline: 28.0838x; 28.0838x over previous
import functools

import jax
import jax.numpy as jnp
from jax import lax
from jax.experimental import pallas as pl
from jax.experimental.pallas import tpu as pltpu
from jax.experimental.pallas import tpu_sc as plsc

_LANES = 16
_CORES = 2
_SUBCORES = 16
_TILES = _CORES * _SUBCORES
_CHUNK = 128
_ROW = 8
_TC_BLK = 2048


def _build_table_tc(xp, pp, W, b, head_w, n_pad):
    grid = n_pad // _TC_BLK

    def body(x_ref, pos_ref, w_ref, b_ref, hw_ref, tab_ref):
        w2 = jnp.dot(w_ref[...], hw_ref[...])
        cc = jnp.dot(b_ref[...], hw_ref[...])
        g = jnp.dot(x_ref[...], w2) + cc
        z = jnp.zeros((_TC_BLK, 3), jnp.float32)
        tab_ref[...] = jnp.concatenate(
            [pos_ref[...], x_ref[...][:, 1:3], g, z], axis=1)

    return pl.pallas_call(
        body,
        grid=(grid,),
        in_specs=[
            pl.BlockSpec((_TC_BLK, 4), lambda i: (i, 0)),
            pl.BlockSpec((_TC_BLK, 2), lambda i: (i, 0)),
            pl.BlockSpec((4, 64), lambda i: (0, 0)),
            pl.BlockSpec((1, 64), lambda i: (0, 0)),
            pl.BlockSpec((64, 1), lambda i: (0, 0)),
        ],
        out_specs=pl.BlockSpec((_TC_BLK, _ROW), lambda i: (i, 0)),
        out_shape=jax.ShapeDtypeStruct((n_pad, _ROW), jnp.float32),
    )(xp, pp, W, b.reshape(1, 64), head_w)


def _sc_flux(table, srcp, dstp, n_pad, e_pad):
    mesh = plsc.VectorSubcoreMesh(
        core_axis_name="c", subcore_axis_name="s",
        num_cores=_CORES, num_subcores=_SUBCORES)
    ept = e_pad // _TILES
    k_chunks = ept // _CHUNK
    npt = n_pad // _SUBCORES

    @functools.partial(
        pl.kernel,
        out_type=jax.ShapeDtypeStruct((_CORES, n_pad), jnp.float32),
        mesh=mesh,
        compiler_params=pltpu.CompilerParams(
            needs_layout_passes=False, use_tc_tiling_on_sc=False),
        scratch_types=[
            pltpu.VMEM((_CHUNK,), jnp.int32),
            pltpu.VMEM((_CHUNK,), jnp.int32),
            pltpu.VMEM((_CHUNK, _ROW), jnp.float32),
            pltpu.VMEM((_CHUNK, _ROW), jnp.float32),
            pltpu.VMEM((_CHUNK,), jnp.float32),
            pltpu.VMEM((_CHUNK,), jnp.float32),
            pltpu.VMEM((npt,), jnp.float32),
            pltpu.VMEM_SHARED((n_pad,), jnp.float32),
            pltpu.SemaphoreType.DMA,
            pltpu.SemaphoreType.DMA,
        ],
    )
    def k(tab_hbm, src_hbm, dst_hbm, out_hbm,
          sidx, didx, srows, drows, mpos, mneg, obuf, acc, sem1, sem2):
        cid = lax.axis_index("c")
        sid = lax.axis_index("s")
        wid = cid * _SUBCORES + sid

        zero16 = jnp.zeros((_LANES,), jnp.float32)

        def zbody(j, carry):
            obuf[pl.ds(j * _LANES, _LANES)] = zero16
            return carry

        lax.fori_loop(0, npt // _LANES, zbody, 0)
        pltpu.sync_copy(obuf, acc.at[pl.ds(sid * npt, npt)])
        plsc.subcore_barrier()

        base0 = wid * ept
        lane = lax.iota(jnp.int32, _LANES)
        cols = [jnp.full((_LANES,), c, jnp.int32) for c in range(5)]

        def chunk_body(i, carry):
            base = base0 + i * _CHUNK
            pltpu.sync_copy(src_hbm.at[pl.ds(base, _CHUNK)], sidx)
            pltpu.sync_copy(dst_hbm.at[pl.ds(base, _CHUNK)], didx)
            pltpu.async_copy(tab_hbm.at[sidx], srows, sem1).wait()
            pltpu.async_copy(tab_hbm.at[didx], drows, sem2).wait()
            for gg in range(_CHUNK // _LANES):
                rows = lane + gg * _LANES
                px_s = plsc.load_gather(srows, [rows, cols[0]])
                py_s = plsc.load_gather(srows, [rows, cols[1]])
                u_s = plsc.load_gather(srows, [rows, cols[2]])
                v_s = plsc.load_gather(srows, [rows, cols[3]])
                g_s = plsc.load_gather(srows, [rows, cols[4]])
                px_d = plsc.load_gather(drows, [rows, cols[0]])
                py_d = plsc.load_gather(drows, [rows, cols[1]])
                dx = px_d - px_s
                dy = py_d - py_s
                r2 = dx * dx + dy * dy
                num = jnp.maximum(u_s * dx + v_s * dy, 0.0)
                yi = (plsc.bitcast(r2, jnp.int32) >> 1) + 0x1FBD1DF5
                y = plsc.bitcast(yi, jnp.float32)
                y = 0.5 * (y + r2 / y)
                y = 0.5 * (y + r2 / y)
                y = 0.5 * (y + r2 / y)
                dist = y + 1e-6
                m = num / (dist * dist) * g_s
                mpos[pl.ds(gg * _LANES, _LANES)] = m
                mneg[pl.ds(gg * _LANES, _LANES)] = -m
            pltpu.sync_copy(mpos, acc.at[didx], add=True)
            pltpu.sync_copy(mneg, acc.at[sidx], add=True)
            return carry

        lax.fori_loop(0, k_chunks, chunk_body, 0)
        plsc.subcore_barrier()

        pltpu.sync_copy(acc.at[pl.ds(sid * npt, npt)], obuf)
        pltpu.sync_copy(obuf, out_hbm.at[cid, pl.ds(sid * npt, npt)])

    return k(table, srcp, dstp)


def _epilogue_tc(partials, x0p, head_b, n_pad):
    rows = n_pad // 128
    p3 = partials.reshape(_CORES, rows, 128)
    x3 = x0p.reshape(rows, 128)
    hb = head_b.reshape(1, 1)

    def body(p_ref, x_ref, hb_ref, pm_ref, dl_ref):
        d = p_ref[0] + p_ref[1] + hb_ref[...]
        dl_ref[...] = d
        pm_ref[...] = jax.nn.softplus(x_ref[...] + d)

    return pl.pallas_call(
        body,
        out_shape=(jax.ShapeDtypeStruct((rows, 128), jnp.float32),
                   jax.ShapeDtypeStruct((rows, 128), jnp.float32)),
    )(p3, x3, hb)


def kernel(x, edge_index, pos, W, b, head_w, head_b):
    n = x.shape[0]
    e = edge_index.shape[1]
    n_pad = -(-n // _TC_BLK) * _TC_BLK
    e_pad = -(-e // (_TILES * _CHUNK)) * (_TILES * _CHUNK)

    pad_id = n_pad - 1
    srcp = jnp.concatenate(
        [edge_index[0], jnp.full((e_pad - e,), pad_id, jnp.int32)])
    dstp = jnp.concatenate(
        [edge_index[1], jnp.full((e_pad - e,), pad_id, jnp.int32)])
    xp = jnp.pad(x, ((0, n_pad - n), (0, 0)))
    pp = jnp.pad(pos, ((0, n_pad - n), (0, 0)))

    table = _build_table_tc(xp, pp, W, b, head_w, n_pad)
    partials = _sc_flux(table, srcp, dstp, n_pad, e_pad)
    pm_p, dl_p = _epilogue_tc(partials, xp[:, 0], head_b, n_pad)

    pm10_next = pm_p.reshape(n_pad)[:n][:, None]
    delta_pm10 = dl_p.reshape(n_pad)[:n][:, None]
    return (pm10_next, delta_pm10)


# speedup vs baseline: 66.9480x; 2.3839x over previous
import functools

import jax
import jax.numpy as jnp
from jax import lax
from jax.experimental import pallas as pl
from jax.experimental.pallas import tpu as pltpu
from jax.experimental.pallas import tpu_sc as plsc

_LANES = 16
_CORES = 2
_SUBCORES = 16
_TILES = _CORES * _SUBCORES
_CHUNK = 128
_ROW = 8
_TC_BLK = 2048


def _build_table_tc(xp, pp, W, b, head_w, n_pad):
    grid = n_pad // _TC_BLK

    def body(x_ref, pos_ref, w_ref, b_ref, hw_ref, tab_ref):
        w2 = jnp.dot(w_ref[...], hw_ref[...])
        cc = jnp.dot(b_ref[...], hw_ref[...])
        g = jnp.dot(x_ref[...], w2) + cc
        z = jnp.zeros((_TC_BLK, 3), jnp.float32)
        tab_ref[...] = jnp.concatenate(
            [pos_ref[...], x_ref[...][:, 1:3], g, z], axis=1)

    return pl.pallas_call(
        body,
        grid=(grid,),
        in_specs=[
            pl.BlockSpec((_TC_BLK, 4), lambda i: (i, 0)),
            pl.BlockSpec((_TC_BLK, 2), lambda i: (i, 0)),
            pl.BlockSpec((4, 64), lambda i: (0, 0)),
            pl.BlockSpec((1, 64), lambda i: (0, 0)),
            pl.BlockSpec((64, 1), lambda i: (0, 0)),
        ],
        out_specs=pl.BlockSpec((_TC_BLK, _ROW), lambda i: (i, 0)),
        out_shape=jax.ShapeDtypeStruct((n_pad, _ROW), jnp.float32),
    )(xp, pp, W, b.reshape(1, 64), head_w)


_NBUF = 6


def _sc_flux(table, posp, srcp, dstp, n_pad, e_pad):
    mesh = plsc.VectorSubcoreMesh(
        core_axis_name="c", subcore_axis_name="s",
        num_cores=_CORES, num_subcores=_SUBCORES)
    ept = e_pad // _TILES
    k_chunks = ept // _CHUNK
    npt = n_pad // _SUBCORES

    @functools.partial(
        pl.kernel,
        out_type=jax.ShapeDtypeStruct((_CORES, n_pad), jnp.float32),
        mesh=mesh,
        compiler_params=pltpu.CompilerParams(
            needs_layout_passes=False, use_tc_tiling_on_sc=False),
        scratch_types=[
            [pltpu.VMEM((_CHUNK,), jnp.int32) for _ in range(_NBUF)],
            [pltpu.VMEM((_CHUNK,), jnp.int32) for _ in range(_NBUF)],
            [pltpu.VMEM((_CHUNK, _ROW), jnp.float32) for _ in range(_NBUF)],
            [pltpu.VMEM((_CHUNK, 2), jnp.float32) for _ in range(_NBUF)],
            [pltpu.VMEM((_CHUNK,), jnp.float32) for _ in range(_NBUF)],
            [pltpu.VMEM((_CHUNK,), jnp.float32) for _ in range(_NBUF)],
            pltpu.VMEM((npt,), jnp.float32),
            pltpu.VMEM_SHARED((n_pad,), jnp.float32),
            [pltpu.SemaphoreType.DMA for _ in range(_NBUF)],
            [pltpu.SemaphoreType.DMA for _ in range(_NBUF)],
            [pltpu.SemaphoreType.DMA for _ in range(_NBUF)],
        ],
    )
    def k(tab_hbm, pos_hbm, src_hbm, dst_hbm, out_hbm,
          sidx, didx, srows, drows, mpos, mneg, obuf, acc,
          semi, semr, sems):
        cid = lax.axis_index("c")
        sid = lax.axis_index("s")
        wid = cid * _SUBCORES + sid

        zero16 = jnp.zeros((_LANES,), jnp.float32)

        def zbody(j, carry):
            obuf[pl.ds(j * _LANES, _LANES)] = zero16
            return carry

        lax.fori_loop(0, npt // _LANES, zbody, 0)
        pltpu.sync_copy(obuf, acc.at[pl.ds(sid * npt, npt)])
        plsc.subcore_barrier()

        base0 = wid * ept
        lane = lax.iota(jnp.int32, _LANES)
        cols = [jnp.full((_LANES,), c, jnp.int32) for c in range(5)]

        def fire_idx(j, s):
            base = base0 + j * _CHUNK
            pltpu.async_copy(src_hbm.at[pl.ds(base, _CHUNK)], sidx[s], semi[s])
            pltpu.async_copy(dst_hbm.at[pl.ds(base, _CHUNK)], didx[s], semi[s])

        def wait_idx(s):
            pltpu.make_async_copy(
                src_hbm.at[pl.ds(0, _CHUNK)], sidx[s], semi[s]).wait()
            pltpu.make_async_copy(
                dst_hbm.at[pl.ds(0, _CHUNK)], didx[s], semi[s]).wait()

        def fire_rows(s):
            pltpu.async_copy(tab_hbm.at[sidx[s]], srows[s], semr[s])
            pltpu.async_copy(pos_hbm.at[didx[s]], drows[s], semr[s])

        def wait_rows(s):
            pltpu.make_async_copy(tab_hbm.at[sidx[s]], srows[s], semr[s]).wait()
            pltpu.make_async_copy(pos_hbm.at[didx[s]], drows[s], semr[s]).wait()

        def fire_scat(s):
            pltpu.async_copy(mpos[s], acc.at[didx[s]], sems[s], add=True)
            pltpu.async_copy(mneg[s], acc.at[sidx[s]], sems[s], add=True)

        def wait_scat(s):
            pltpu.make_async_copy(mpos[s], acc.at[didx[s]], sems[s]).wait()
            pltpu.make_async_copy(mneg[s], acc.at[sidx[s]], sems[s]).wait()

        def compute(s):
            for gg in range(_CHUNK // _LANES):
                rows = lane + gg * _LANES
                px_s = plsc.load_gather(srows[s], [rows, cols[0]])
                py_s = plsc.load_gather(srows[s], [rows, cols[1]])
                u_s = plsc.load_gather(srows[s], [rows, cols[2]])
                v_s = plsc.load_gather(srows[s], [rows, cols[3]])
                g_s = plsc.load_gather(srows[s], [rows, cols[4]])
                px_d = plsc.load_gather(drows[s], [rows, cols[0]])
                py_d = plsc.load_gather(drows[s], [rows, cols[1]])
                dx = px_d - px_s
                dy = py_d - py_s
                r2 = dx * dx + dy * dy
                num = jnp.maximum(u_s * dx + v_s * dy, 0.0)
                yi = (plsc.bitcast(r2, jnp.int32) >> 1) + 0x1FBD1DF5
                y = plsc.bitcast(yi, jnp.float32)
                y = 0.5 * (y + r2 / y)
                y = 0.5 * (y + r2 / y)
                y = 0.5 * (y + r2 / y)
                dist = y + 1e-6
                m = num / (dist * dist) * g_s
                mpos[s][pl.ds(gg * _LANES, _LANES)] = m
                mneg[s][pl.ds(gg * _LANES, _LANES)] = -m

        for j in range(4):
            fire_idx(jnp.int32(j), j)
        wait_idx(0)
        fire_rows(0)
        wait_idx(1)
        fire_rows(1)

        def outer(i0, carry):
            for b in range(_NBUF):
                i = i0 * _NBUF + b
                s = b
                s1 = (b + 2) % _NBUF
                s2 = (b + 4) % _NBUF
                wait_rows(s)

                @pl.when(i + 2 < k_chunks)
                def _():
                    wait_idx(s1)
                    fire_rows(s1)

                @pl.when(jnp.logical_and(i >= 2, i + 4 < k_chunks))
                def _():
                    wait_scat(s2)

                @pl.when(i + 4 < k_chunks)
                def _():
                    fire_idx(i + 4, s2)

                compute(s)
                fire_scat(s)
            return carry

        lax.fori_loop(0, k_chunks // _NBUF, outer, 0)
        for s in range(_NBUF):
            wait_scat(s)
        plsc.subcore_barrier()

        pltpu.sync_copy(acc.at[pl.ds(sid * npt, npt)], obuf)
        pltpu.sync_copy(obuf, out_hbm.at[cid, pl.ds(sid * npt, npt)])

    return k(table, posp, srcp, dstp)


def _epilogue_tc(partials, x0p, head_b, n_pad):
    rows = n_pad // 128
    p3 = partials.reshape(_CORES, rows, 128)
    x3 = x0p.reshape(rows, 128)
    hb = head_b.reshape(1, 1)

    def body(p_ref, x_ref, hb_ref, pm_ref, dl_ref):
        d = p_ref[0] + p_ref[1] + hb_ref[...]
        dl_ref[...] = d
        pm_ref[...] = jax.nn.softplus(x_ref[...] + d)

    return pl.pallas_call(
        body,
        out_shape=(jax.ShapeDtypeStruct((rows, 128), jnp.float32),
                   jax.ShapeDtypeStruct((rows, 128), jnp.float32)),
    )(p3, x3, hb)


def kernel(x, edge_index, pos, W, b, head_w, head_b):
    n = x.shape[0]
    e = edge_index.shape[1]
    n_pad = -(-n // _TC_BLK) * _TC_BLK
    e_quant = _TILES * _CHUNK * _NBUF
    e_pad = -(-e // e_quant) * e_quant

    pad_id = n_pad - 1
    srcp = jnp.concatenate(
        [edge_index[0], jnp.full((e_pad - e,), pad_id, jnp.int32)])
    dstp = jnp.concatenate(
        [edge_index[1], jnp.full((e_pad - e,), pad_id, jnp.int32)])
    xp = jnp.pad(x, ((0, n_pad - n), (0, 0)))
    pp = jnp.pad(pos, ((0, n_pad - n), (0, 0)))

    table = _build_table_tc(xp, pp, W, b, head_w, n_pad)
    partials = _sc_flux(table, pp, srcp, dstp, n_pad, e_pad)
    pm_p, dl_p = _epilogue_tc(partials, xp[:, 0], head_b, n_pad)

    pm10_next = pm_p.reshape(n_pad)[:n][:, None]
    delta_pm10 = dl_p.reshape(n_pad)[:n][:, None]
    return (pm10_next, delta_pm10)


# speedup vs baseline: 70.5638x; 1.0540x over previous
import functools

import jax
import jax.numpy as jnp
from jax import lax
from jax.experimental import pallas as pl
from jax.experimental.pallas import tpu as pltpu
from jax.experimental.pallas import tpu_sc as plsc

_LANES = 16
_CORES = 2
_SUBCORES = 16
_TILES = _CORES * _SUBCORES
_CHUNK = 128
_ROW = 8
_TC_BLK = 2048
_NBUF = 6


def _build_table_tc(xp, pp, W, b, head_w, n_pad):
    grid = n_pad // _TC_BLK

    def body(x_ref, pos_ref, w_ref, b_ref, hw_ref, tab_ref):
        w2 = jnp.dot(w_ref[...], hw_ref[...])
        cc = jnp.dot(b_ref[...], hw_ref[...])
        g = jnp.dot(x_ref[...], w2) + cc
        z = jnp.zeros((_TC_BLK, 3), jnp.float32)
        tab_ref[...] = jnp.concatenate(
            [pos_ref[...], x_ref[...][:, 1:3], g, z], axis=1)

    return pl.pallas_call(
        body,
        grid=(grid,),
        in_specs=[
            pl.BlockSpec((_TC_BLK, 4), lambda i: (i, 0)),
            pl.BlockSpec((_TC_BLK, 2), lambda i: (i, 0)),
            pl.BlockSpec((4, 64), lambda i: (0, 0)),
            pl.BlockSpec((1, 64), lambda i: (0, 0)),
            pl.BlockSpec((64, 1), lambda i: (0, 0)),
        ],
        out_specs=pl.BlockSpec((_TC_BLK, _ROW), lambda i: (i, 0)),
        out_shape=jax.ShapeDtypeStruct((n_pad, _ROW), jnp.float32),
    )(xp, pp, W, b.reshape(1, 64), head_w)


def _sc_flux(table, pos, ei, n_pad):
    e = ei.shape[1]
    assert e % _CHUNK == 0
    n_chunks = e // _CHUNK
    bc, rem = divmod(n_chunks, _TILES)
    assert bc >= _NBUF
    k_max = bc + (1 if rem else 0)
    outer_n = -(-k_max // _NBUF)

    mesh = plsc.VectorSubcoreMesh(
        core_axis_name="c", subcore_axis_name="s",
        num_cores=_CORES, num_subcores=_SUBCORES)

    @functools.partial(
        pl.kernel,
        out_type=jax.ShapeDtypeStruct((_TILES, n_pad), jnp.float32),
        mesh=mesh,
        compiler_params=pltpu.CompilerParams(
            needs_layout_passes=False, use_tc_tiling_on_sc=False),
        scratch_types=[
            [pltpu.VMEM((_CHUNK,), jnp.int32) for _ in range(_NBUF)],
            [pltpu.VMEM((_CHUNK,), jnp.int32) for _ in range(_NBUF)],
            [pltpu.VMEM((_CHUNK, _ROW), jnp.float32) for _ in range(_NBUF)],
            [pltpu.VMEM((_CHUNK, 2), jnp.float32) for _ in range(_NBUF)],
            pltpu.VMEM((n_pad,), jnp.float32),
            [pltpu.SemaphoreType.DMA for _ in range(_NBUF)],
            [pltpu.SemaphoreType.DMA for _ in range(_NBUF)],
        ],
    )
    def k(tab_hbm, pos_hbm, ei_hbm, out_hbm,
          sidx, didx, srows, drows, acc, semi, semr):
        cid = lax.axis_index("c")
        sid = lax.axis_index("s")
        wid = cid * _SUBCORES + sid
        cnt = bc + jnp.where(wid < rem, 1, 0)
        base0 = (wid * bc + jnp.minimum(wid, rem)) * _CHUNK

        zero64 = jnp.zeros((_LANES,), jnp.float32)

        def zbody(j, carry):
            base = j * 4 * _LANES
            acc[pl.ds(base, _LANES)] = zero64
            acc[pl.ds(base + _LANES, _LANES)] = zero64
            acc[pl.ds(base + 2 * _LANES, _LANES)] = zero64
            acc[pl.ds(base + 3 * _LANES, _LANES)] = zero64
            return carry

        lax.fori_loop(0, n_pad // (4 * _LANES), zbody, 0)

        lane = lax.iota(jnp.int32, _LANES)
        cols = [jnp.full((_LANES,), c, jnp.int32) for c in range(5)]

        def fire_idx(j, s):
            base = base0 + j * _CHUNK
            pltpu.async_copy(ei_hbm.at[0, pl.ds(base, _CHUNK)], sidx[s], semi[s])
            pltpu.async_copy(ei_hbm.at[1, pl.ds(base, _CHUNK)], didx[s], semi[s])

        def wait_idx(s):
            pltpu.make_async_copy(
                ei_hbm.at[0, pl.ds(0, _CHUNK)], sidx[s], semi[s]).wait()
            pltpu.make_async_copy(
                ei_hbm.at[1, pl.ds(0, _CHUNK)], didx[s], semi[s]).wait()

        def fire_rows(s):
            pltpu.async_copy(tab_hbm.at[sidx[s]], srows[s], semr[s])
            pltpu.async_copy(pos_hbm.at[didx[s]], drows[s], semr[s])

        def wait_rows(s):
            pltpu.make_async_copy(tab_hbm.at[sidx[s]], srows[s], semr[s]).wait()
            pltpu.make_async_copy(pos_hbm.at[didx[s]], drows[s], semr[s]).wait()

        def compute(s):
            for gg in range(_CHUNK // _LANES):
                rows = lane + gg * _LANES
                px_s = plsc.load_gather(srows[s], [rows, cols[0]])
                py_s = plsc.load_gather(srows[s], [rows, cols[1]])
                u_s = plsc.load_gather(srows[s], [rows, cols[2]])
                v_s = plsc.load_gather(srows[s], [rows, cols[3]])
                g_s = plsc.load_gather(srows[s], [rows, cols[4]])
                px_d = plsc.load_gather(drows[s], [rows, cols[0]])
                py_d = plsc.load_gather(drows[s], [rows, cols[1]])
                dx = px_d - px_s
                dy = py_d - py_s
                r2 = dx * dx + dy * dy
                num = jnp.maximum(u_s * dx + v_s * dy, 0.0)
                yi = (plsc.bitcast(r2, jnp.int32) >> 1) + 0x1FBD1DF5
                y = plsc.bitcast(yi, jnp.float32)
                y = 0.5 * (y + r2 / y)
                y = 0.5 * (y + r2 / y)
                y = 0.5 * (y + r2 / y)
                dist = y + 1e-6
                m = num / (dist * dist) * g_s
                d16 = didx[s][pl.ds(gg * _LANES, _LANES)]
                s16 = sidx[s][pl.ds(gg * _LANES, _LANES)]
                plsc.addupdate_scatter(acc, [d16], m)
                plsc.addupdate_scatter(acc, [s16], -m)

        for j in range(4):
            fire_idx(j, j)
        wait_idx(0)
        fire_rows(0)
        wait_idx(1)
        fire_rows(1)

        def outer(i0, carry):
            for b in range(_NBUF):
                i = i0 * _NBUF + b
                s = b
                s1 = (b + 2) % _NBUF
                s2 = (b + 4) % _NBUF

                @pl.when(i + 2 < cnt)
                def _():
                    wait_idx(s1)
                    fire_rows(s1)

                @pl.when(i + 4 < cnt)
                def _():
                    fire_idx(i + 4, s2)

                @pl.when(i < cnt)
                def _():
                    wait_rows(s)
                    compute(s)
            return carry

        lax.fori_loop(0, outer_n, outer, 0)

        pltpu.sync_copy(acc, out_hbm.at[wid])

    return k(table, pos, ei)


def _epilogue_tc(partials, x0p, head_b, n_pad):
    rows = n_pad // 128
    p3 = partials.reshape(_TILES, rows, 128)
    x3 = x0p.reshape(rows, 128)
    hb = head_b.reshape(1, 1)

    def body(p_ref, x_ref, hb_ref, pm_ref, dl_ref):
        d = jnp.sum(p_ref[...], axis=0) + hb_ref[...]
        dl_ref[...] = d
        pm_ref[...] = jax.nn.softplus(x_ref[...] + d)

    return pl.pallas_call(
        body,
        out_shape=(jax.ShapeDtypeStruct((rows, 128), jnp.float32),
                   jax.ShapeDtypeStruct((rows, 128), jnp.float32)),
    )(p3, x3, hb)


def kernel(x, edge_index, pos, W, b, head_w, head_b):
    n = x.shape[0]
    n_pad = -(-n // _TC_BLK) * _TC_BLK

    xp = jnp.pad(x, ((0, n_pad - n), (0, 0)))
    pp = jnp.pad(pos, ((0, n_pad - n), (0, 0)))

    table = _build_table_tc(xp, pp, W, b, head_w, n_pad)
    partials = _sc_flux(table, pp, edge_index, n_pad)
    pm_p, dl_p = _epilogue_tc(partials, xp[:, 0], head_b, n_pad)

    pm10_next = pm_p.reshape(n_pad)[:n][:, None]
    delta_pm10 = dl_p.reshape(n_pad)[:n][:, None]
    return (pm10_next, delta_pm10)
